# Initial kernel scaffold; baseline (speedup 1.0000x reference)
import jax
import jax.numpy as jnp
from jax.experimental import pallas as pl


def kernel(x, fMin, fMax):
    raise NotImplementedError("write your pallas kernel here")


# speedup vs baseline: 3130.4621x; 3130.4621x over previous
import functools

import jax
import jax.numpy as jnp
from jax import lax
from jax.experimental import pallas as pl
from jax.experimental.pallas import tpu as pltpu
from jax.experimental.pallas import tpu_sc as plsc

NBINS = 256
L = 16
NC = 2
NS = 16
NW = NC * NS


def _make_sc_bucketize(n: int, chunk: int):
    assert n % (NW * chunk) == 0
    per_w = n // NW
    nchunk = per_w // chunk
    vpc = chunk // L

    mesh = plsc.VectorSubcoreMesh(core_axis_name="c", subcore_axis_name="s")

    @functools.partial(
        pl.kernel,
        mesh=mesh,
        out_type=jax.ShapeDtypeStruct((n,), jnp.int32),
        scratch_types=[
            pltpu.VMEM((chunk,), jnp.float32),
            pltpu.VMEM((chunk,), jnp.int32),
            pltpu.VMEM((64,), jnp.float32),
        ],
    )
    def sc_bucketize(x_hbm, consts_hbm, y_hbm, inb, outb, cv):
        wid = lax.axis_index("s") * NC + lax.axis_index("c")
        base = wid * per_w

        pltpu.sync_copy(consts_hbm, cv)
        scale = cv[pl.ds(0, L)]
        beta = cv[pl.ds(L, L)]
        fmin_v = cv[pl.ds(2 * L, L)]
        fmax_v = cv[pl.ds(3 * L, L)]

        def chunk_body(g, carry):
            off = pl.multiple_of(base + g * chunk, 8)
            pltpu.sync_copy(x_hbm.at[pl.ds(off, chunk)], inb)

            def vec_body(i, c):
                v = inb[pl.ds(i * L, L)]
                t = v * scale + beta
                k = t.astype(jnp.int32)
                k = jnp.minimum(k, jnp.int32(NBINS - 1))
                k = jnp.where(v >= fmax_v, jnp.int32(NBINS), k)
                k = jnp.where(v < fmin_v, jnp.int32(0), k)
                outb[pl.ds(i * L, L)] = k
                return c

            lax.fori_loop(0, vpc, vec_body, 0)
            pltpu.sync_copy(outb, y_hbm.at[pl.ds(off, chunk)])
            return carry

        lax.fori_loop(0, nchunk, chunk_body, 0)

    return sc_bucketize


def kernel(x, fMin, fMax):
    n = x.size
    xf = x.reshape(n)
    scale = jnp.float32(NBINS - 1) / (fMax - fMin)
    beta = jnp.float32(1.0) - fMin * scale
    consts = jnp.concatenate([
        jnp.full((L,), scale, jnp.float32),
        jnp.full((L,), beta, jnp.float32),
        jnp.full((L,), fMin, jnp.float32),
        jnp.full((L,), fMax, jnp.float32),
    ])
    y = _make_sc_bucketize(n, 32768)(xf, consts)
    return y.reshape(x.shape).astype(jnp.int64)


# speedup vs baseline: 4063.3316x; 1.2980x over previous
import functools

import jax
import jax.numpy as jnp
from jax import lax
from jax.experimental import pallas as pl
from jax.experimental.pallas import tpu as pltpu
from jax.experimental.pallas import tpu_sc as plsc

NBINS = 256
L = 16
NC = 2
NS = 16
NW = NC * NS
UNROLL = 4


def _make_sc_bucketize(n: int, chunk: int):
    assert n % (NW * chunk) == 0
    per_w = n // NW
    nchunk = per_w // chunk
    assert nchunk % 2 == 0
    pairs = nchunk // 2
    vpc = chunk // (L * UNROLL)

    mesh = plsc.VectorSubcoreMesh(core_axis_name="c", subcore_axis_name="s")

    @functools.partial(
        pl.kernel,
        mesh=mesh,
        out_type=jax.ShapeDtypeStruct((n,), jnp.int32),
        scratch_types=[
            pltpu.VMEM((chunk,), jnp.float32),
            pltpu.VMEM((chunk,), jnp.float32),
            pltpu.VMEM((chunk,), jnp.int32),
            pltpu.VMEM((chunk,), jnp.int32),
            pltpu.VMEM((2 * L,), jnp.float32),
            pltpu.SemaphoreType.DMA,
            pltpu.SemaphoreType.DMA,
            pltpu.SemaphoreType.DMA,
            pltpu.SemaphoreType.DMA,
        ],
    )
    def sc_bucketize(x_hbm, consts_hbm, y_hbm,
                     in0, in1, out0, out1, cv, is0, is1, os0, os1):
        wid = lax.axis_index("s") * NC + lax.axis_index("c")
        base = wid * per_w

        pltpu.sync_copy(consts_hbm, cv)
        scale = cv[pl.ds(0, L)]
        beta = cv[pl.ds(L, L)]
        zero = jnp.zeros((L,), jnp.int32)
        top = jnp.full((L,), NBINS, jnp.int32)

        def compute(src, dst):
            def vec_body(i, c):
                for u in range(UNROLL):
                    o = i * (L * UNROLL) + u * L
                    v = src[pl.ds(o, L)]
                    t = v * scale + beta
                    k = t.astype(jnp.int32)
                    k = jnp.minimum(k, top)
                    k = jnp.maximum(k, zero)
                    dst[pl.ds(o, L)] = k
                return c
            lax.fori_loop(0, vpc, vec_body, 0)

        def start_in(c, buf, sem):
            off = pl.multiple_of(base + c * chunk, 8)
            pltpu.async_copy(x_hbm.at[pl.ds(off, chunk)], buf, sem)

        def wait_in(buf, sem):
            pltpu.make_async_copy(x_hbm.at[pl.ds(0, chunk)], buf, sem).wait()

        def start_out(buf, c, sem):
            off = pl.multiple_of(base + c * chunk, 8)
            pltpu.async_copy(buf, y_hbm.at[pl.ds(off, chunk)], sem)

        def wait_out(buf, sem):
            pltpu.make_async_copy(buf, y_hbm.at[pl.ds(0, chunk)], sem).wait()

        start_in(0, in0, is0)

        def pair_body(t, carry):
            c0 = 2 * t

            start_in(c0 + 1, in1, is1)
            wait_in(in0, is0)

            @pl.when(t > 0)
            def _():
                wait_out(out0, os0)

            compute(in0, out0)
            start_out(out0, c0, os0)

            @pl.when(t < pairs - 1)
            def _():
                start_in(c0 + 2, in0, is0)

            wait_in(in1, is1)

            @pl.when(t > 0)
            def _():
                wait_out(out1, os1)

            compute(in1, out1)
            start_out(out1, c0 + 1, os1)
            return carry

        lax.fori_loop(0, pairs, pair_body, 0)
        wait_out(out0, os0)
        wait_out(out1, os1)

    return sc_bucketize


def kernel(x, fMin, fMax):
    n = x.size
    xf = x.reshape(n)
    scale = jnp.float32(NBINS - 1) / (fMax - fMin)
    beta = jnp.float32(1.0) - fMin * scale
    consts = jnp.concatenate([
        jnp.full((L,), scale, jnp.float32),
        jnp.full((L,), beta, jnp.float32),
    ])
    y = _make_sc_bucketize(n, 16384)(xf, consts)
    return y.reshape(x.shape).astype(jnp.int64)


# speedup vs baseline: 4198.8551x; 1.0334x over previous
import functools

import jax
import jax.numpy as jnp
from jax import lax
from jax.experimental import pallas as pl
from jax.experimental.pallas import tpu as pltpu
from jax.experimental.pallas import tpu_sc as plsc

NBINS = 256
L = 16
NC = 2
NS = 16
NW = NC * NS
UNROLL = 8


def _make_sc_bucketize(n: int, chunk: int):
    assert n % (NW * chunk) == 0
    per_w = n // NW
    nchunk = per_w // chunk
    assert nchunk % 2 == 0
    pairs = nchunk // 2

    mesh = plsc.VectorSubcoreMesh(core_axis_name="c", subcore_axis_name="s")

    @functools.partial(
        pl.kernel,
        mesh=mesh,
        out_type=jax.ShapeDtypeStruct((n,), jnp.int32),
        scratch_types=[
            pltpu.VMEM((chunk,), jnp.float32),
            pltpu.VMEM((chunk,), jnp.float32),
            pltpu.VMEM((chunk,), jnp.int32),
            pltpu.VMEM((chunk,), jnp.int32),
            pltpu.VMEM((2 * L,), jnp.float32),
            pltpu.SemaphoreType.DMA,
            pltpu.SemaphoreType.DMA,
            pltpu.SemaphoreType.DMA,
            pltpu.SemaphoreType.DMA,
        ],
    )
    def sc_bucketize(x_hbm, consts_hbm, y_hbm,
                     in0, in1, out0, out1, cv, is0, is1, os0, os1):
        wid = lax.axis_index("s") * NC + lax.axis_index("c")
        base = wid * per_w

        pltpu.sync_copy(consts_hbm, cv)
        scale = cv[pl.ds(0, L)]
        beta = cv[pl.ds(L, L)]
        zero = jnp.zeros((L,), jnp.int32)
        top = jnp.full((L,), NBINS, jnp.int32)

        def compute(src, dst):
            @plsc.parallel_loop(0, chunk, step=L, unroll=UNROLL)
            def _(o):
                v = src[pl.ds(o, L)]
                t = v * scale + beta
                k = t.astype(jnp.int32)
                k = jnp.minimum(k, top)
                k = jnp.maximum(k, zero)
                dst[pl.ds(o, L)] = k

        def start_in(c, buf, sem):
            off = pl.multiple_of(base + c * chunk, 8)
            pltpu.async_copy(x_hbm.at[pl.ds(off, chunk)], buf, sem)

        def wait_in(buf, sem):
            pltpu.make_async_copy(x_hbm.at[pl.ds(0, chunk)], buf, sem).wait()

        def start_out(buf, c, sem):
            off = pl.multiple_of(base + c * chunk, 8)
            pltpu.async_copy(buf, y_hbm.at[pl.ds(off, chunk)], sem)

        def wait_out(buf, sem):
            pltpu.make_async_copy(buf, y_hbm.at[pl.ds(0, chunk)], sem).wait()

        start_in(0, in0, is0)

        def pair_body(t, carry):
            c0 = 2 * t

            start_in(c0 + 1, in1, is1)
            wait_in(in0, is0)

            @pl.when(t > 0)
            def _():
                wait_out(out0, os0)

            compute(in0, out0)
            start_out(out0, c0, os0)

            @pl.when(t < pairs - 1)
            def _():
                start_in(c0 + 2, in0, is0)

            wait_in(in1, is1)

            @pl.when(t > 0)
            def _():
                wait_out(out1, os1)

            compute(in1, out1)
            start_out(out1, c0 + 1, os1)
            return carry

        lax.fori_loop(0, pairs, pair_body, 0)
        wait_out(out0, os0)
        wait_out(out1, os1)

    return sc_bucketize


def kernel(x, fMin, fMax):
    n = x.size
    xf = x.reshape(n)
    scale = jnp.float32(NBINS - 1) / (fMax - fMin)
    beta = jnp.float32(1.0) - fMin * scale
    consts = jnp.concatenate([
        jnp.full((L,), scale, jnp.float32),
        jnp.full((L,), beta, jnp.float32),
    ])
    y = _make_sc_bucketize(n, 16384)(xf, consts)
    return y.reshape(x.shape).astype(jnp.int64)


# speedup vs baseline: 4243.2505x; 1.0106x over previous
import functools

import jax
import jax.numpy as jnp
from jax import lax
from jax.experimental import pallas as pl
from jax.experimental.pallas import tpu as pltpu
from jax.experimental.pallas import tpu_sc as plsc

NBINS = 256
L = 16
NC = 2
NS = 16
NW = NC * NS
UNROLL = 8
NBUF = 4
CHUNK = 8192


def _make_sc_bucketize(n: int, chunk: int, nbuf: int):
    assert n % (NW * chunk) == 0
    per_w = n // NW
    nchunk = per_w // chunk
    assert nchunk % nbuf == 0
    rounds = nchunk // nbuf

    mesh = plsc.VectorSubcoreMesh(core_axis_name="c", subcore_axis_name="s")

    @functools.partial(
        pl.kernel,
        mesh=mesh,
        out_type=jax.ShapeDtypeStruct((n,), jnp.int32),
        scratch_types=(
            [pltpu.VMEM((chunk,), jnp.float32) for _ in range(nbuf)]
            + [pltpu.VMEM((chunk,), jnp.int32) for _ in range(nbuf)]
            + [pltpu.VMEM((2 * L,), jnp.float32)]
            + [pltpu.SemaphoreType.DMA for _ in range(2 * nbuf)]
        ),
    )
    def sc_bucketize(x_hbm, consts_hbm, y_hbm, *bufs):
        inb = bufs[:nbuf]
        outb = bufs[nbuf:2 * nbuf]
        cv = bufs[2 * nbuf]
        isem = bufs[2 * nbuf + 1:2 * nbuf + 1 + nbuf]
        osem = bufs[2 * nbuf + 1 + nbuf:2 * nbuf + 1 + 2 * nbuf]

        wid = lax.axis_index("s") * NC + lax.axis_index("c")
        base = wid * per_w

        pltpu.sync_copy(consts_hbm, cv)
        scale = cv[pl.ds(0, L)]
        beta = cv[pl.ds(L, L)]
        zero = jnp.zeros((L,), jnp.int32)
        top = jnp.full((L,), NBINS, jnp.int32)

        def compute(src, dst):
            @plsc.parallel_loop(0, chunk, step=L, unroll=UNROLL)
            def _(o):
                v = src[pl.ds(o, L)]
                t = v * scale + beta
                k = t.astype(jnp.int32)
                k = jnp.minimum(k, top)
                k = jnp.maximum(k, zero)
                dst[pl.ds(o, L)] = k

        def start_in(c, b):
            off = pl.multiple_of(base + c * chunk, 8)
            pltpu.async_copy(x_hbm.at[pl.ds(off, chunk)], inb[b], isem[b])

        def wait_in(b):
            pltpu.make_async_copy(
                x_hbm.at[pl.ds(0, chunk)], inb[b], isem[b]).wait()

        def start_out(b, c):
            off = pl.multiple_of(base + c * chunk, 8)
            pltpu.async_copy(outb[b], y_hbm.at[pl.ds(off, chunk)], osem[b])

        def wait_out(b):
            pltpu.make_async_copy(
                outb[b], y_hbm.at[pl.ds(0, chunk)], osem[b]).wait()

        for b in range(nbuf):
            start_in(b, b)

        def round_body(q, carry):
            for b in range(nbuf):
                c = q * nbuf + b
                wait_in(b)

                @pl.when(q > 0)
                def _():
                    wait_out(b)

                compute(inb[b], outb[b])
                start_out(b, c)

                @pl.when(q < rounds - 1)
                def _():
                    start_in(c + nbuf, b)
            return carry

        lax.fori_loop(0, rounds, round_body, 0)
        for b in range(nbuf):
            wait_out(b)

    return sc_bucketize


def kernel(x, fMin, fMax):
    n = x.size
    xf = x.reshape(n)
    scale = jnp.float32(NBINS - 1) / (fMax - fMin)
    beta = jnp.float32(1.0) - fMin * scale
    consts = jnp.concatenate([
        jnp.full((L,), scale, jnp.float32),
        jnp.full((L,), beta, jnp.float32),
    ])
    y = _make_sc_bucketize(n, CHUNK, NBUF)(xf, consts)
    return y.reshape(x.shape).astype(jnp.int64)
